# baseline (speedup 1.0000x reference)
import functools

import jax
import jax.numpy as jnp
from jax import lax
from jax.experimental import pallas as pl
from jax.experimental.pallas import tpu as pltpu
from jax.experimental.pallas import tpu_sc as plsc

ROWS = 64
COLS = 100000
CH = 20000
NCH = COLS // CH
L = 16
NW = 32
RPW = ROWS // NW
BIG = 2**31 - 1
NEG = float("-inf")

_LN2 = 0.6931471805599453
_SQRT2 = 1.4142135623730951


def _vlog(x):
    xi = lax.bitcast_convert_type(x, jnp.int32)
    e = (xi >> 23) - 127
    mf = lax.bitcast_convert_type((xi & 0x007FFFFF) | 0x3F800000, jnp.float32)
    big = mf > _SQRT2
    mf = jnp.where(big, mf * 0.5, mf)
    e = jnp.where(big, e + 1, e)
    t = (mf - 1.0) / (mf + 1.0)
    t2 = t * t
    p = 1.0 + t2 * (1.0 / 3.0 + t2 * (1.0 / 5.0 + t2 * (1.0 / 7.0 + t2 / 9.0)))
    return e.astype(jnp.float32) * _LN2 + 2.0 * t * p


def _sc_body(logits_hbm, actions_hbm, lp_hbm, mode_hbm,
             act_v, buf0, buf1, olp_v, omode_v, sem0, sem1):
    c = lax.axis_index("c")
    s = lax.axis_index("s")
    wid = c * 16 + s
    lanes = jnp.arange(16, dtype=jnp.int32)

    pltpu.sync_copy(actions_hbm, act_v)

    bufs = (buf0, buf1)
    sems = (sem0, sem1)

    olp = jnp.zeros((L,), jnp.float32)
    omode = jnp.zeros((L,), jnp.int32)

    for rj in range(RPW):
        r = wid * RPW + rj
        rw = (r >> 4) << 4
        avec = act_v[pl.ds(rw, L)].astype(jnp.float32)
        a_r = jnp.sum(jnp.where(lanes == (r - rw), avec, 0.0)).astype(jnp.int32)

        vm = jnp.full((L,), NEG, jnp.float32)
        vi = jnp.full((L,), BIG, jnp.int32)
        vs = jnp.zeros((L,), jnp.float32)
        gv = jnp.float32(0.0)

        rbase = pl.multiple_of(r * COLS, 8)
        cp = pltpu.async_copy(
            logits_hbm.at[pl.ds(rbase, CH)], bufs[0], sems[0])
        for ci in range(NCH):
            if ci + 1 < NCH:
                cpn = pltpu.async_copy(
                    logits_hbm.at[pl.ds(rbase + (ci + 1) * CH, CH)],
                    bufs[(ci + 1) % 2], sems[(ci + 1) % 2])
            cp.wait()
            buf = bufs[ci % 2]
            base = ci * CH

            def ibody(j, carry, buf=buf, base=base):
                vm, vi, vs = carry
                x = buf[pl.ds(j * L, L)]
                col = (base + j * L) + lanes
                nm = jnp.maximum(vm, x)
                vs = vs * jnp.exp(vm - nm) + jnp.exp(x - nm)
                vi = jnp.where(x > vm, col, vi)
                return nm, vi, vs

            vm, vi, vs = lax.fori_loop(0, CH // L, ibody, (vm, vi, vs))

            off0 = a_r - base
            w = jnp.clip(off0, 0, CH - L)
            v = buf[pl.ds(w, L)]
            g = jnp.sum(jnp.where(lanes == (off0 - w), v, 0.0))
            gv = jnp.where((off0 >= 0) & (off0 < CH), g, gv)
            if ci + 1 < NCH:
                cp = cpn

        m = jnp.max(vm)
        mb = jnp.full((L,), m)
        stot = jnp.sum(vs * jnp.exp(vm - mb))
        idx = jnp.min(jnp.where(vm == mb, vi, BIG))
        lp = gv - m - jnp.max(_vlog(jnp.full((L,), stot)))
        olp = jnp.where(lanes == rj, lp, olp)
        omode = jnp.where(lanes == rj, idx, omode)

    olp_v[...] = olp
    omode_v[...] = omode
    obase = pl.multiple_of(wid * L, 8)
    pltpu.sync_copy(olp_v, lp_hbm.at[pl.ds(obase, L)])
    pltpu.sync_copy(omode_v, mode_hbm.at[pl.ds(obase, L)])


@jax.jit
def _sc_call(logits, actions_flat):
    mesh = plsc.VectorSubcoreMesh(core_axis_name="c", subcore_axis_name="s")
    f = functools.partial(
        pl.kernel,
        mesh=mesh,
        out_type=[
            jax.ShapeDtypeStruct((NW * L,), jnp.float32),
            jax.ShapeDtypeStruct((NW * L,), jnp.int32),
        ],
        scratch_types=[
            pltpu.VMEM((ROWS,), jnp.int32),
            pltpu.VMEM((CH,), jnp.float32),
            pltpu.VMEM((CH,), jnp.float32),
            pltpu.VMEM((L,), jnp.float32),
            pltpu.VMEM((L,), jnp.int32),
            pltpu.SemaphoreType.DMA,
            pltpu.SemaphoreType.DMA,
        ],
        compiler_params=pltpu.CompilerParams(needs_layout_passes=False),
    )(_sc_body)
    return f(logits, actions_flat)


def kernel(logits, actions):
    a = actions.reshape(-1).astype(jnp.int32)
    lp_pad, mode_pad = _sc_call(logits.reshape(-1), a)
    lp = lp_pad.reshape(NW, L)[:, :RPW].reshape(ROWS, 1)
    mode = mode_pad.reshape(NW, L)[:, :RPW].reshape(ROWS, 1)
    return lp, mode
